# Initial kernel scaffold; baseline (speedup 1.0000x reference)
import jax
import jax.numpy as jnp
from jax.experimental import pallas as pl


def kernel(query, history_bevs, reference_points, spatial_shapes, level_start_index, pos_embedding, params):
    raise NotImplementedError("write your pallas kernel here")


# speedup vs baseline: 1.0000x; 1.0000x over previous
import jax
import jax.numpy as jnp
from jax.experimental import pallas as pl

B = 1
C = 256
NH = 8
DH = C // NH
NL = 4
NP_ = 4
HGRID = 64
NQ = HGRID * HGRID
NV = NL * NQ
WF = 4


def _layer_norm(x, g, b):
    m = jnp.mean(x, axis=-1, keepdims=True)
    v = jnp.mean((x - m) ** 2, axis=-1, keepdims=True)
    return (x - m) / jnp.sqrt(v + 1e-5) * g + b


def _grid_sample_bilinear(value, grid):
    Bn, Cc, H, W = value.shape
    Hq, Wq = grid.shape[1], grid.shape[2]
    gx = (grid[..., 0] + 1.0) * W / 2.0 - 0.5
    gy = (grid[..., 1] + 1.0) * H / 2.0 - 0.5
    x0 = jnp.floor(gx); y0 = jnp.floor(gy)
    x1 = x0 + 1.0; y1 = y0 + 1.0
    wx1 = gx - x0; wx0 = 1.0 - wx1
    wy1 = gy - y0; wy0 = 1.0 - wy1
    vflat = value.reshape(Bn, Cc, H * W)

    def gather(xi, yi):
        valid = (xi >= 0) & (xi <= W - 1) & (yi >= 0) & (yi <= H - 1)
        xc = jnp.clip(xi, 0, W - 1).astype(jnp.int32)
        yc = jnp.clip(yi, 0, H - 1).astype(jnp.int32)
        idx = (yc * W + xc).reshape(Bn, 1, Hq * Wq)
        idx = jnp.broadcast_to(idx, (Bn, Cc, Hq * Wq))
        g = jnp.take_along_axis(vflat, idx, axis=2).reshape(Bn, Cc, Hq, Wq)
        return g * valid.reshape(Bn, 1, Hq, Wq).astype(value.dtype)

    out = (gather(x0, y0) * (wx0 * wy0)[:, None]
           + gather(x1, y0) * (wx1 * wy0)[:, None]
           + gather(x0, y1) * (wx0 * wy1)[:, None]
           + gather(x1, y1) * (wx1 * wy1)[:, None])
    return out


def _msda(value, sampling_locations, attention_weights):
    bs, nv, nh, d = value.shape
    nq = sampling_locations.shape[1]
    L = NL
    P = sampling_locations.shape[4]
    grids = 2.0 * sampling_locations - 1.0
    sampled = []
    start = 0
    for lvl in range(L):
        H = HGRID; W = HGRID
        v = value[:, start:start + H * W]
        start += H * W
        v = jnp.transpose(v, (0, 2, 3, 1)).reshape(bs * nh, d, H, W)
        g = jnp.transpose(grids[:, :, :, lvl], (0, 2, 1, 3, 4)).reshape(bs * nh, nq, P, 2)
        sampled.append(_grid_sample_bilinear(v, g))
    aw = jnp.transpose(attention_weights, (0, 2, 1, 3, 4)).reshape(bs * nh, 1, nq, L * P)
    st = jnp.stack(sampled, axis=-2).reshape(bs * nh, d, nq, L * P)
    out = (st * aw).sum(-1).reshape(bs, nh * d, nq)
    return jnp.transpose(out, (0, 2, 1))


def _self_attention(x, Wqkv, Wo, bo):
    b, n, _ = x.shape
    qkv = x @ Wqkv
    q, k, v = jnp.split(qkv, 3, axis=-1)

    def heads(t):
        return jnp.transpose(t.reshape(b, n, NH, DH), (0, 2, 1, 3))

    q, k, v = heads(q), heads(k), heads(v)
    dots = jnp.einsum('bhid,bhjd->bhij', q, k) * (DH ** -0.5)
    attn = jax.nn.softmax(dots, axis=-1)
    out = jnp.einsum('bhij,bhjd->bhid', attn, v)
    out = jnp.transpose(out, (0, 2, 1, 3)).reshape(b, n, NH * DH)
    return out @ Wo + bo


def _add_kernel(a_ref, b_ref, o_ref):
    o_ref[...] = a_ref[...] + b_ref[...]


def _pallas_add(a, b):
    return pl.pallas_call(
        _add_kernel,
        out_shape=jax.ShapeDtypeStruct(a.shape, a.dtype),
    )(a, b)


def kernel(query, history_bevs, reference_points, spatial_shapes, level_start_index, pos_embedding, params):
    p = params
    xin = query + pos_embedding
    x = _self_attention(_layer_norm(xin, p['ln1_g'], p['ln1_b']), p['Wqkv'], p['Wo'], p['bo']) + query
    xq = _layer_norm(x + pos_embedding, p['ln2_g'], p['ln2_b'])
    vn = _layer_norm(history_bevs, p['ln2_g'], p['ln2_b'])

    b, nq, _ = xq.shape
    nv = vn.shape[1]
    value = (vn @ p['Wv'] + p['bv']).reshape(b, nv, NH, DH)
    so = (xq @ p['Wso'] + p['bso']).reshape(b, nq, NH, NL, NP_, 2)
    aw = (xq @ p['Wa'] + p['ba']).reshape(b, nq, NH, NL * NP_)
    aw = jax.nn.softmax(aw, axis=-1).reshape(b, nq, NH, NL, NP_)
    norm = jnp.stack([spatial_shapes[:, 1].astype(jnp.float32), spatial_shapes[:, 0].astype(jnp.float32)], -1)
    loc = reference_points[:, :, None, :, None, :] + so / norm[None, None, None, :, None, :]
    co = _msda(value, loc, aw)
    x = co @ p['Wout'] + p['bout'] + x

    xn = _layer_norm(x, p['ln3_g'], p['ln3_b'])
    ff = (xn @ p['W1'] + p['b1']) @ p['W2'] + p['b2']
    return _pallas_add(ff, x)


# speedup vs baseline: 481.7307x; 481.7282x over previous
import functools

import jax
import jax.numpy as jnp
from jax import lax
from jax.experimental import pallas as pl
from jax.experimental.pallas import tpu as pltpu
from jax.experimental.pallas import tpu_sc as plsc

C = 256
NH = 8
DH = C // NH
NL = 4
NP_ = 4
HGRID = 64
NQ = HGRID * HGRID
NV = NL * NQ
WF = 4

QBLK = 512
NQB = NQ // QBLK

NPAIR = NQ * NH
NCONTRIB = NL * NP_ * 4

NTILE = 32
NPT = NPAIR // NTILE
GB = 16
NBLK = NPT // GB


def _ln(x, g, b):
    m = jnp.mean(x, axis=-1, keepdims=True)
    v = jnp.mean((x - m) ** 2, axis=-1, keepdims=True)
    return (x - m) / jnp.sqrt(v + 1e-5) * g + b


def _qkv_body(x_ref, g_ref, b_ref, w_ref, o_ref):
    xn = _ln(x_ref[...], g_ref[...], b_ref[...])
    o_ref[...] = jnp.dot(xn, w_ref[...], preferred_element_type=jnp.float32)


def _tc_qkv(xin, g, b, w):
    return pl.pallas_call(
        _qkv_body,
        grid=(NQB,),
        in_specs=[
            pl.BlockSpec((QBLK, C), lambda i: (i, 0)),
            pl.BlockSpec((1, C), lambda i: (0, 0)),
            pl.BlockSpec((1, C), lambda i: (0, 0)),
            pl.BlockSpec((C, 3 * C), lambda i: (0, 0)),
        ],
        out_specs=pl.BlockSpec((QBLK, 3 * C), lambda i: (i, 0)),
        out_shape=jax.ShapeDtypeStruct((NQ, 3 * C), jnp.float32),
    )(xin, g, b, w)


def _attn_body(q_ref, k_ref, v_ref, o_ref):
    q = q_ref[0]
    k = k_ref[0]
    s = lax.dot_general(q, k, (((1,), (1,)), ((), ())),
                        preferred_element_type=jnp.float32) * (DH ** -0.5)
    m = jnp.max(s, axis=-1, keepdims=True)
    e = jnp.exp(s - m)
    z = jnp.sum(e, axis=-1, keepdims=True)
    a = e / z
    o_ref[0] = jnp.dot(a, v_ref[0], preferred_element_type=jnp.float32)


def _tc_attn(qh, kh, vh):
    return pl.pallas_call(
        _attn_body,
        grid=(NH, NQB),
        in_specs=[
            pl.BlockSpec((1, QBLK, DH), lambda h, i: (h, i, 0)),
            pl.BlockSpec((1, NQ, DH), lambda h, i: (h, 0, 0)),
            pl.BlockSpec((1, NQ, DH), lambda h, i: (h, 0, 0)),
        ],
        out_specs=pl.BlockSpec((1, QBLK, DH), lambda h, i: (h, i, 0)),
        out_shape=jax.ShapeDtypeStruct((NH, NQ, DH), jnp.float32),
    )(qh, kh, vh)


def _proj_res_body(a_ref, w_ref, b_ref, r_ref, o_ref):
    o_ref[...] = (jnp.dot(a_ref[...], w_ref[...], preferred_element_type=jnp.float32)
                  + b_ref[...] + r_ref[...])


def _tc_proj_res(a, w, b, res):
    return pl.pallas_call(
        _proj_res_body,
        grid=(NQB,),
        in_specs=[
            pl.BlockSpec((QBLK, C), lambda i: (i, 0)),
            pl.BlockSpec((C, C), lambda i: (0, 0)),
            pl.BlockSpec((1, C), lambda i: (0, 0)),
            pl.BlockSpec((QBLK, C), lambda i: (i, 0)),
        ],
        out_specs=pl.BlockSpec((QBLK, C), lambda i: (i, 0)),
        out_shape=jax.ShapeDtypeStruct((NQ, C), jnp.float32),
    )(a, w, b, res)


def _value_body(h_ref, g_ref, b_ref, w_ref, bv_ref, o_ref):
    xn = _ln(h_ref[...], g_ref[...], b_ref[...])
    res = jnp.dot(xn, w_ref[...], preferred_element_type=jnp.float32) + bv_ref[...]
    for h in range(NH):
        o_ref[h, 0] = res[:, h * DH:(h + 1) * DH]


def _tc_value(hist, g, b, w, bv):
    blk = 1024
    return pl.pallas_call(
        _value_body,
        grid=(NV // blk,),
        in_specs=[
            pl.BlockSpec((blk, C), lambda i: (i, 0)),
            pl.BlockSpec((1, C), lambda i: (0, 0)),
            pl.BlockSpec((1, C), lambda i: (0, 0)),
            pl.BlockSpec((C, C), lambda i: (0, 0)),
            pl.BlockSpec((1, C), lambda i: (0, 0)),
        ],
        out_specs=pl.BlockSpec((NH, 1, blk, DH), lambda i: (0, i // 4, i % 4, 0)),
        out_shape=jax.ShapeDtypeStruct((NH, NL, NQ, DH), jnp.float32),
    )(hist, g, b, w, bv)


def _corner_body(v_ref, o_ref):
    v = v_ref[0, 0].reshape(HGRID, HGRID, DH)
    sx = jnp.concatenate([v[:, 1:, :], v[:, HGRID - 1:, :]], axis=1)
    sy = jnp.concatenate([v[1:, :, :], v[HGRID - 1:, :, :]], axis=0)
    sxy = jnp.concatenate([sx[1:, :, :], sx[HGRID - 1:, :, :]], axis=0)
    o_ref[0, 0] = jnp.concatenate([v, sx, sy, sxy], axis=-1).reshape(NQ, 4 * DH)


def _tc_corner_pack(vplanes):
    return pl.pallas_call(
        _corner_body,
        grid=(NH, NL),
        in_specs=[pl.BlockSpec((1, 1, NQ, DH), lambda h, l: (h, l, 0, 0))],
        out_specs=pl.BlockSpec((1, 1, NQ, 4 * DH), lambda h, l: (h, l, 0, 0)),
        out_shape=jax.ShapeDtypeStruct((NH, NL, NQ, 4 * DH), jnp.float32),
    )(vplanes)


def _samp_body(x_ref, g_ref, b_ref, wso_ref, bso_ref, wa_ref, ba_ref, so_ref, aw_ref):
    xq = _ln(x_ref[...], g_ref[...], b_ref[...])
    so_ref[...] = (jnp.dot(xq, wso_ref[...], preferred_element_type=jnp.float32)
                   + bso_ref[...])
    logits = (jnp.dot(xq, wa_ref[...], preferred_element_type=jnp.float32)
              + ba_ref[...])
    parts = []
    for h in range(NH):
        blk = logits[:, h * 16:(h + 1) * 16]
        m = jnp.max(blk, axis=-1, keepdims=True)
        e = jnp.exp(blk - m)
        parts.append(e / jnp.sum(e, axis=-1, keepdims=True))
    aw_ref[...] = jnp.concatenate(parts, axis=-1)


def _tc_samp(xqin, g, b, wso, bso, wa, ba):
    nso = NH * NL * NP_ * 2
    naw = NH * NL * NP_
    return pl.pallas_call(
        _samp_body,
        grid=(NQB,),
        in_specs=[
            pl.BlockSpec((QBLK, C), lambda i: (i, 0)),
            pl.BlockSpec((1, C), lambda i: (0, 0)),
            pl.BlockSpec((1, C), lambda i: (0, 0)),
            pl.BlockSpec((C, nso), lambda i: (0, 0)),
            pl.BlockSpec((1, nso), lambda i: (0, 0)),
            pl.BlockSpec((C, naw), lambda i: (0, 0)),
            pl.BlockSpec((1, naw), lambda i: (0, 0)),
        ],
        out_specs=[
            pl.BlockSpec((QBLK, nso), lambda i: (i, 0)),
            pl.BlockSpec((QBLK, naw), lambda i: (i, 0)),
        ],
        out_shape=[
            jax.ShapeDtypeStruct((NQ, nso), jnp.float32),
            jax.ShapeDtypeStruct((NQ, naw), jnp.float32),
        ],
    )(xqin, g, b, wso, bso, wa, ba)


def _outffn_body(s_ref, wout_ref, bout_ref, x_ref, g_ref, b_ref,
                 w1_ref, b1_ref, w2_ref, b2_ref, o_ref):
    x2 = (jnp.dot(s_ref[...], wout_ref[...], preferred_element_type=jnp.float32)
          + bout_ref[...] + x_ref[...])
    xn = _ln(x2, g_ref[...], b_ref[...])
    h1 = jnp.dot(xn, w1_ref[...], preferred_element_type=jnp.float32) + b1_ref[...]
    ff = jnp.dot(h1, w2_ref[...], preferred_element_type=jnp.float32) + b2_ref[...]
    o_ref[...] = ff + x2


def _tc_outffn(sampled, wout, bout, x, g, b, w1, b1, w2, b2):
    return pl.pallas_call(
        _outffn_body,
        grid=(NQB,),
        in_specs=[
            pl.BlockSpec((QBLK, C), lambda i: (i, 0)),
            pl.BlockSpec((C, C), lambda i: (0, 0)),
            pl.BlockSpec((1, C), lambda i: (0, 0)),
            pl.BlockSpec((QBLK, C), lambda i: (i, 0)),
            pl.BlockSpec((1, C), lambda i: (0, 0)),
            pl.BlockSpec((1, C), lambda i: (0, 0)),
            pl.BlockSpec((C, WF * C), lambda i: (0, 0)),
            pl.BlockSpec((1, WF * C), lambda i: (0, 0)),
            pl.BlockSpec((WF * C, C), lambda i: (0, 0)),
            pl.BlockSpec((1, C), lambda i: (0, 0)),
        ],
        out_specs=pl.BlockSpec((QBLK, C), lambda i: (i, 0)),
        out_shape=jax.ShapeDtypeStruct((NQ, C), jnp.float32),
    )(sampled, wout, bout, x, g, b, w1, b1, w2, b2)


def _expand_idx_w(so, aw, ref):
    so5 = so.reshape(NQ, NH, NL, NP_, 2)
    aw4 = aw.reshape(NQ, NH, NL, NP_)
    loc = ref[:, None, :, None, :] + so5 / float(HGRID)
    gx = loc[..., 0] * HGRID - 0.5
    gy = loc[..., 1] * HGRID - 0.5
    x0 = jnp.floor(gx)
    y0 = jnp.floor(gy)
    wx1 = gx - x0
    wx0 = 1.0 - wx1
    wy1 = gy - y0
    wy0 = 1.0 - wy1
    bx = jnp.clip(x0, 0.0, HGRID - 2.0)
    by = jnp.clip(y0, 0.0, HGRID - 2.0)
    lvl = jnp.arange(NL, dtype=jnp.int32)[None, None, :, None]
    hh = jnp.arange(NH, dtype=jnp.int32)[None, :, None, None]
    idx = ((hh * NL + lvl) * NQ
           + by.astype(jnp.int32) * HGRID + bx.astype(jnp.int32))
    ws = []
    for dy in (0.0, 1.0):
        for dx in (0.0, 1.0):
            sx = bx + dx
            sy = by + dy
            fx = jnp.where(sx == x0, wx0, jnp.where(sx == x0 + 1.0, wx1, 0.0))
            fy = jnp.where(sy == y0, wy0, jnp.where(sy == y0 + 1.0, wy1, 0.0))
            ws.append(fx * fy * aw4)
    w = jnp.stack(ws, axis=-1)
    return idx.reshape(NPAIR, NL * NP_), w.reshape(NPAIR, NCONTRIB)


def _sc_body(value_hbm, idx_hbm, w_hbm, out_hbm, idx_v, w_v, rows_v, out_v,
             sem_i, sem_w, sem_r):
    wid = lax.axis_index("s") * 2 + lax.axis_index("c")
    brow = wid * NBLK

    def fire_idx(b, slot):
        pltpu.async_copy(idx_hbm.at[brow + b], idx_v.at[slot], sem_i)
        pltpu.async_copy(w_hbm.at[brow + b], w_v.at[slot], sem_w)

    def wait_idx(slot):
        pltpu.make_async_copy(idx_hbm.at[0], idx_v.at[slot], sem_i).wait()
        pltpu.make_async_copy(w_hbm.at[0], w_v.at[slot], sem_w).wait()

    def fire_gathers(slot):
        def fj(j, c):
            pltpu.async_copy(value_hbm.at[idx_v.at[slot, pl.ds(j * 16, 16)]],
                             rows_v.at[slot, pl.ds(j * 16, 16)], sem_r)
            return c
        lax.fori_loop(0, GB, fj, 0)

    def drain_gathers(slot):
        def dj(j, c):
            pltpu.make_async_copy(
                value_hbm.at[idx_v.at[slot, pl.ds(j * 16, 16)]],
                rows_v.at[slot, pl.ds(j * 16, 16)], sem_r).wait()
            return c
        lax.fori_loop(0, GB, dj, 0)

    def compute_block(b, slot):
        dnums = lax.GatherDimensionNumbers(
            offset_dims=(), collapsed_slice_dims=(0,), start_index_map=(0,))

        def pj(j, c):
            wvecs = [w_v[slot, pl.ds(j * 64 + g * 16, 16)] for g in range(4)]
            acc0 = jnp.zeros((16,), jnp.float32)
            acc1 = jnp.zeros((16,), jnp.float32)
            for i in range(NCONTRIB):
                g, lane = divmod(i, 16)
                ws = lax.gather(
                    wvecs[g], jnp.full((16, 1), lane, jnp.int32), dnums, (1,),
                    mode=lax.GatherScatterMode.PROMISE_IN_BOUNDS)
                r0 = rows_v[slot, j * 16 + i // 4, pl.ds((i % 4) * DH, 16)]
                r1 = rows_v[slot, j * 16 + i // 4, pl.ds((i % 4) * DH + 16, 16)]
                acc0 = acc0 + ws * r0
                acc1 = acc1 + ws * r1
            pit = b * GB + j
            out_v[pit // 4, pl.ds((pit % 4) * DH, 16)] = acc0
            out_v[pit // 4, pl.ds((pit % 4) * DH + 16, 16)] = acc1
            return c
        lax.fori_loop(0, GB, pj, 0)

    def body_seq(b, c):
        fire_idx(b, 0)
        wait_idx(0)
        fire_gathers(0)
        drain_gathers(0)
        compute_block(b, 0)
        return c

    lax.fori_loop(0, NBLK, body_seq, 0)
    pltpu.sync_copy(out_v, out_hbm.at[pl.ds(wid * (NPT // 4), NPT // 4)])


def _sc_gather(valtab, idx, w):
    mesh = plsc.VectorSubcoreMesh(core_axis_name="c", subcore_axis_name="s")
    fn = functools.partial(
        pl.kernel,
        out_type=jax.ShapeDtypeStruct((NPAIR // 4, 4 * DH), jnp.float32),
        mesh=mesh,
        scratch_types=[
            pltpu.VMEM((2, GB * 16), jnp.int32),
            pltpu.VMEM((2, GB * NCONTRIB), jnp.float32),
            pltpu.VMEM((2, GB * 16, 4 * DH), jnp.float32),
            pltpu.VMEM((NPT // 4, 4 * DH), jnp.float32),
            pltpu.SemaphoreType.DMA,
            pltpu.SemaphoreType.DMA,
            pltpu.SemaphoreType.DMA,
        ],
    )(_sc_body)
    return fn(valtab, idx.reshape(NPAIR // GB, GB * 16),
              w.reshape(NPAIR // GB, GB * NCONTRIB))


def kernel(query, history_bevs, reference_points, spatial_shapes,
           level_start_index, pos_embedding, params):
    p = params
    q2 = query[0]
    pos2 = pos_embedding[0]
    hist2 = history_bevs[0]

    def r2(v):
        return v.reshape(1, -1)

    qkv = _tc_qkv(q2 + pos2, r2(p['ln1_g']), r2(p['ln1_b']), p['Wqkv'])
    qkvh = jnp.transpose(qkv.reshape(NQ, 3 * NH, DH), (1, 0, 2))
    attnh = _tc_attn(qkvh[:NH], qkvh[NH:2 * NH], qkvh[2 * NH:])
    attnout = jnp.transpose(attnh, (1, 0, 2)).reshape(NQ, C)
    x = _tc_proj_res(attnout, p['Wo'], r2(p['bo']), q2)

    vplanes = _tc_value(hist2, r2(p['ln2_g']), r2(p['ln2_b']), p['Wv'],
                        r2(p['bv']))
    table4 = _tc_corner_pack(vplanes).reshape(NH * NL * NQ, 4 * DH)

    so, aw = _tc_samp(x + pos2, r2(p['ln2_g']), r2(p['ln2_b']),
                      p['Wso'], r2(p['bso']), p['Wa'], r2(p['ba']))
    idx, w = _expand_idx_w(so, aw, reference_points[0])

    sampled = _sc_gather(table4, idx, w).reshape(NQ, C)

    out = _tc_outffn(sampled, p['Wout'], r2(p['bout']), x,
                     r2(p['ln3_g']), r2(p['ln3_b']),
                     p['W1'], r2(p['b1']), p['W2'], r2(p['b2']))
    return out[None]


# speedup vs baseline: 528.2195x; 1.0965x over previous
import functools

import jax
import jax.numpy as jnp
from jax import lax
from jax.experimental import pallas as pl
from jax.experimental.pallas import tpu as pltpu
from jax.experimental.pallas import tpu_sc as plsc

C = 256
NH = 8
DH = C // NH
NL = 4
NP_ = 4
HGRID = 64
NQ = HGRID * HGRID
NV = NL * NQ
WF = 4

QBLK = 512
NQB = NQ // QBLK

NPAIR = NQ * NH
NCONTRIB = NL * NP_ * 4

NTILE = 32
NPT = NPAIR // NTILE
GB = 16
NBLK = NPT // GB


def _ln(x, g, b):
    m = jnp.mean(x, axis=-1, keepdims=True)
    v = jnp.mean((x - m) ** 2, axis=-1, keepdims=True)
    return (x - m) / jnp.sqrt(v + 1e-5) * g + b


def _qkv_body(x_ref, g_ref, b_ref, w_ref, o_ref):
    xn = _ln(x_ref[...], g_ref[...], b_ref[...])
    res = jnp.dot(xn, w_ref[...], preferred_element_type=jnp.float32)
    for k in range(3 * NH):
        o_ref[k] = res[:, k * DH:(k + 1) * DH]


def _tc_qkv(xin, g, b, w):
    return pl.pallas_call(
        _qkv_body,
        grid=(NQB,),
        in_specs=[
            pl.BlockSpec((QBLK, C), lambda i: (i, 0)),
            pl.BlockSpec((1, C), lambda i: (0, 0)),
            pl.BlockSpec((1, C), lambda i: (0, 0)),
            pl.BlockSpec((C, 3 * C), lambda i: (0, 0)),
        ],
        out_specs=pl.BlockSpec((3 * NH, QBLK, DH), lambda i: (0, i, 0)),
        out_shape=jax.ShapeDtypeStruct((3 * NH, NQ, DH), jnp.float32),
    )(xin, g, b, w)


def _attn_body(q_ref, k_ref, v_ref, o_ref):
    q = q_ref[0]
    k = k_ref[0]
    s = lax.dot_general(q, k, (((1,), (1,)), ((), ())),
                        preferred_element_type=jnp.float32) * (DH ** -0.5)
    m = jnp.max(s, axis=-1, keepdims=True)
    e = jnp.exp(s - m)
    z = jnp.sum(e, axis=-1, keepdims=True)
    a = e / z
    o_ref[0] = jnp.dot(a, v_ref[0], preferred_element_type=jnp.float32)


def _tc_attn(qh, kh, vh):
    return pl.pallas_call(
        _attn_body,
        grid=(NH, NQB),
        in_specs=[
            pl.BlockSpec((1, QBLK, DH), lambda h, i: (h, i, 0)),
            pl.BlockSpec((1, NQ, DH), lambda h, i: (h, 0, 0)),
            pl.BlockSpec((1, NQ, DH), lambda h, i: (h, 0, 0)),
        ],
        out_specs=pl.BlockSpec((1, QBLK, DH), lambda h, i: (h, i, 0)),
        out_shape=jax.ShapeDtypeStruct((NH, NQ, DH), jnp.float32),
    )(qh, kh, vh)


def _proj_res_body(a_ref, w_ref, b_ref, r_ref, o_ref):
    a = jnp.concatenate([a_ref[h] for h in range(NH)], axis=-1)
    o_ref[...] = (jnp.dot(a, w_ref[...], preferred_element_type=jnp.float32)
                  + b_ref[...] + r_ref[...])


def _tc_proj_res(attnh, w, b, res):
    return pl.pallas_call(
        _proj_res_body,
        grid=(NQB,),
        in_specs=[
            pl.BlockSpec((NH, QBLK, DH), lambda i: (0, i, 0)),
            pl.BlockSpec((C, C), lambda i: (0, 0)),
            pl.BlockSpec((1, C), lambda i: (0, 0)),
            pl.BlockSpec((QBLK, C), lambda i: (i, 0)),
        ],
        out_specs=pl.BlockSpec((QBLK, C), lambda i: (i, 0)),
        out_shape=jax.ShapeDtypeStruct((NQ, C), jnp.float32),
    )(attnh, w, b, res)


def _value_body(h_ref, g_ref, b_ref, w_ref, bv_ref, o_ref):
    xn = _ln(h_ref[...], g_ref[...], b_ref[...])
    res = jnp.dot(xn, w_ref[...], preferred_element_type=jnp.float32) + bv_ref[...]
    for h in range(NH):
        o_ref[h, 0] = res[:, h * DH:(h + 1) * DH]


def _tc_value(hist, g, b, w, bv):
    blk = 1024
    return pl.pallas_call(
        _value_body,
        grid=(NV // blk,),
        in_specs=[
            pl.BlockSpec((blk, C), lambda i: (i, 0)),
            pl.BlockSpec((1, C), lambda i: (0, 0)),
            pl.BlockSpec((1, C), lambda i: (0, 0)),
            pl.BlockSpec((C, C), lambda i: (0, 0)),
            pl.BlockSpec((1, C), lambda i: (0, 0)),
        ],
        out_specs=pl.BlockSpec((NH, 1, blk, DH), lambda i: (0, i // 4, i % 4, 0)),
        out_shape=jax.ShapeDtypeStruct((NH, NL, NQ, DH), jnp.float32),
    )(hist, g, b, w, bv)


def _corner_body(v_ref, o_ref):
    v = v_ref[0, 0].reshape(HGRID, HGRID, DH)
    sx = jnp.concatenate([v[:, 1:, :], v[:, HGRID - 1:, :]], axis=1)
    sy = jnp.concatenate([v[1:, :, :], v[HGRID - 1:, :, :]], axis=0)
    sxy = jnp.concatenate([sx[1:, :, :], sx[HGRID - 1:, :, :]], axis=0)
    o_ref[0, 0] = jnp.concatenate([v, sx, sy, sxy], axis=-1).reshape(NQ, 4 * DH)


def _tc_corner_pack(vplanes):
    return pl.pallas_call(
        _corner_body,
        grid=(NH, NL),
        in_specs=[pl.BlockSpec((1, 1, NQ, DH), lambda h, l: (h, l, 0, 0))],
        out_specs=pl.BlockSpec((1, 1, NQ, 4 * DH), lambda h, l: (h, l, 0, 0)),
        out_shape=jax.ShapeDtypeStruct((NH, NL, NQ, 4 * DH), jnp.float32),
    )(vplanes)


def _samp_body(x_ref, g_ref, b_ref, wso_ref, bso_ref, wa_ref, ba_ref, so_ref, aw_ref):
    xq = _ln(x_ref[...], g_ref[...], b_ref[...])
    so_ref[...] = (jnp.dot(xq, wso_ref[...], preferred_element_type=jnp.float32)
                   + bso_ref[...])
    logits = (jnp.dot(xq, wa_ref[...], preferred_element_type=jnp.float32)
              + ba_ref[...])
    parts = []
    for h in range(NH):
        blk = logits[:, h * 16:(h + 1) * 16]
        m = jnp.max(blk, axis=-1, keepdims=True)
        e = jnp.exp(blk - m)
        parts.append(e / jnp.sum(e, axis=-1, keepdims=True))
    aw_ref[...] = jnp.concatenate(parts, axis=-1)


def _tc_samp(xqin, g, b, wso, bso, wa, ba):
    nso = NH * NL * NP_ * 2
    naw = NH * NL * NP_
    return pl.pallas_call(
        _samp_body,
        grid=(NQB,),
        in_specs=[
            pl.BlockSpec((QBLK, C), lambda i: (i, 0)),
            pl.BlockSpec((1, C), lambda i: (0, 0)),
            pl.BlockSpec((1, C), lambda i: (0, 0)),
            pl.BlockSpec((C, nso), lambda i: (0, 0)),
            pl.BlockSpec((1, nso), lambda i: (0, 0)),
            pl.BlockSpec((C, naw), lambda i: (0, 0)),
            pl.BlockSpec((1, naw), lambda i: (0, 0)),
        ],
        out_specs=[
            pl.BlockSpec((QBLK, nso), lambda i: (i, 0)),
            pl.BlockSpec((QBLK, naw), lambda i: (i, 0)),
        ],
        out_shape=[
            jax.ShapeDtypeStruct((NQ, nso), jnp.float32),
            jax.ShapeDtypeStruct((NQ, naw), jnp.float32),
        ],
    )(xqin, g, b, wso, bso, wa, ba)


def _outffn_body(s_ref, wout_ref, bout_ref, x_ref, g_ref, b_ref,
                 w1_ref, b1_ref, w2_ref, b2_ref, o_ref):
    x2 = (jnp.dot(s_ref[...], wout_ref[...], preferred_element_type=jnp.float32)
          + bout_ref[...] + x_ref[...])
    xn = _ln(x2, g_ref[...], b_ref[...])
    h1 = jnp.dot(xn, w1_ref[...], preferred_element_type=jnp.float32) + b1_ref[...]
    ff = jnp.dot(h1, w2_ref[...], preferred_element_type=jnp.float32) + b2_ref[...]
    o_ref[...] = ff + x2


def _tc_outffn(sampled, wout, bout, x, g, b, w1, b1, w2, b2):
    return pl.pallas_call(
        _outffn_body,
        grid=(NQB,),
        in_specs=[
            pl.BlockSpec((QBLK, C), lambda i: (i, 0)),
            pl.BlockSpec((C, C), lambda i: (0, 0)),
            pl.BlockSpec((1, C), lambda i: (0, 0)),
            pl.BlockSpec((QBLK, C), lambda i: (i, 0)),
            pl.BlockSpec((1, C), lambda i: (0, 0)),
            pl.BlockSpec((1, C), lambda i: (0, 0)),
            pl.BlockSpec((C, WF * C), lambda i: (0, 0)),
            pl.BlockSpec((1, WF * C), lambda i: (0, 0)),
            pl.BlockSpec((WF * C, C), lambda i: (0, 0)),
            pl.BlockSpec((1, C), lambda i: (0, 0)),
        ],
        out_specs=pl.BlockSpec((QBLK, C), lambda i: (i, 0)),
        out_shape=jax.ShapeDtypeStruct((NQ, C), jnp.float32),
    )(sampled, wout, bout, x, g, b, w1, b1, w2, b2)


def _expand_idx_w(so, aw, ref):
    so5 = so.reshape(NQ, NH, NL, NP_, 2)
    aw4 = aw.reshape(NQ, NH, NL, NP_)
    loc = ref[:, None, :, None, :] + so5 / float(HGRID)
    gx = loc[..., 0] * HGRID - 0.5
    gy = loc[..., 1] * HGRID - 0.5
    x0 = jnp.floor(gx)
    y0 = jnp.floor(gy)
    wx1 = gx - x0
    wx0 = 1.0 - wx1
    wy1 = gy - y0
    wy0 = 1.0 - wy1
    bx = jnp.clip(x0, 0.0, HGRID - 2.0)
    by = jnp.clip(y0, 0.0, HGRID - 2.0)
    lvl = jnp.arange(NL, dtype=jnp.int32)[None, None, :, None]
    hh = jnp.arange(NH, dtype=jnp.int32)[None, :, None, None]
    idx = ((hh * NL + lvl) * NQ
           + by.astype(jnp.int32) * HGRID + bx.astype(jnp.int32))
    ws = []
    for dy in (0.0, 1.0):
        for dx in (0.0, 1.0):
            sx = bx + dx
            sy = by + dy
            fx = jnp.where(sx == x0, wx0, jnp.where(sx == x0 + 1.0, wx1, 0.0))
            fy = jnp.where(sy == y0, wy0, jnp.where(sy == y0 + 1.0, wy1, 0.0))
            ws.append(fx * fy * aw4)
    w = jnp.stack(ws, axis=-1)
    return idx.reshape(NPAIR, NL * NP_), w.reshape(NPAIR, NCONTRIB)


def _sc_body(value_hbm, idx_hbm, w_hbm, out_hbm, idx_v, w_v, rows_v, out_v,
             sem_i, sem_w, sem_r):
    wid = lax.axis_index("s") * 2 + lax.axis_index("c")
    brow = wid * NBLK

    def fire_idx(b, slot):
        pltpu.async_copy(idx_hbm.at[brow + b], idx_v.at[slot], sem_i)
        pltpu.async_copy(w_hbm.at[brow + b], w_v.at[slot], sem_w)

    def wait_idx(slot):
        pltpu.make_async_copy(idx_hbm.at[0], idx_v.at[slot], sem_i).wait()
        pltpu.make_async_copy(w_hbm.at[0], w_v.at[slot], sem_w).wait()

    def fire_gathers(slot):
        def fj(j, c):
            pltpu.async_copy(value_hbm.at[idx_v.at[slot, pl.ds(j * 16, 16)]],
                             rows_v.at[slot, pl.ds(j * 16, 16)], sem_r)
            return c
        lax.fori_loop(0, GB, fj, 0)

    def drain_gathers(slot):
        def dj(j, c):
            pltpu.make_async_copy(
                value_hbm.at[idx_v.at[slot, pl.ds(j * 16, 16)]],
                rows_v.at[slot, pl.ds(j * 16, 16)], sem_r).wait()
            return c
        lax.fori_loop(0, GB, dj, 0)

    def compute_block(b, slot):
        dnums = lax.GatherDimensionNumbers(
            offset_dims=(), collapsed_slice_dims=(0,), start_index_map=(0,))

        def pj(j, c):
            wvecs = [w_v[slot, pl.ds(j * 64 + g * 16, 16)] for g in range(4)]
            acc0 = jnp.zeros((16,), jnp.float32)
            acc1 = jnp.zeros((16,), jnp.float32)
            for i in range(NCONTRIB):
                g, lane = divmod(i, 16)
                ws = lax.gather(
                    wvecs[g], jnp.full((16, 1), lane, jnp.int32), dnums, (1,),
                    mode=lax.GatherScatterMode.PROMISE_IN_BOUNDS)
                r0 = rows_v[slot, j * 16 + i // 4, pl.ds((i % 4) * DH, 16)]
                r1 = rows_v[slot, j * 16 + i // 4, pl.ds((i % 4) * DH + 16, 16)]
                acc0 = acc0 + ws * r0
                acc1 = acc1 + ws * r1
            pit = b * GB + j
            out_v[pit // 4, pl.ds((pit % 4) * DH, 16)] = acc0
            out_v[pit // 4, pl.ds((pit % 4) * DH + 16, 16)] = acc1
            return c
        lax.fori_loop(0, GB, pj, 0)

    def body_seq(b, c):
        fire_idx(b, 0)
        wait_idx(0)
        fire_gathers(0)
        drain_gathers(0)
        compute_block(b, 0)
        return c

    lax.fori_loop(0, NBLK, body_seq, 0)
    pltpu.sync_copy(out_v, out_hbm.at[pl.ds(wid * (NPT // 4), NPT // 4)])


def _sc_gather(valtab, idx, w):
    mesh = plsc.VectorSubcoreMesh(core_axis_name="c", subcore_axis_name="s")
    fn = functools.partial(
        pl.kernel,
        out_type=jax.ShapeDtypeStruct((NPAIR // 4, 4 * DH), jnp.float32),
        mesh=mesh,
        scratch_types=[
            pltpu.VMEM((2, GB * 16), jnp.int32),
            pltpu.VMEM((2, GB * NCONTRIB), jnp.float32),
            pltpu.VMEM((2, GB * 16, 4 * DH), jnp.float32),
            pltpu.VMEM((NPT // 4, 4 * DH), jnp.float32),
            pltpu.SemaphoreType.DMA,
            pltpu.SemaphoreType.DMA,
            pltpu.SemaphoreType.DMA,
        ],
    )(_sc_body)
    return fn(valtab, idx.reshape(NPAIR // GB, GB * 16),
              w.reshape(NPAIR // GB, GB * NCONTRIB))


def kernel(query, history_bevs, reference_points, spatial_shapes,
           level_start_index, pos_embedding, params):
    p = params
    q2 = query[0]
    pos2 = pos_embedding[0]
    hist2 = history_bevs[0]

    def r2(v):
        return v.reshape(1, -1)

    qkvh = _tc_qkv(q2 + pos2, r2(p['ln1_g']), r2(p['ln1_b']), p['Wqkv'])
    attnh = _tc_attn(qkvh[:NH], qkvh[NH:2 * NH], qkvh[2 * NH:])
    x = _tc_proj_res(attnh, p['Wo'], r2(p['bo']), q2)

    vplanes = _tc_value(hist2, r2(p['ln2_g']), r2(p['ln2_b']), p['Wv'],
                        r2(p['bv']))
    table4 = _tc_corner_pack(vplanes).reshape(NH * NL * NQ, 4 * DH)

    so, aw = _tc_samp(x + pos2, r2(p['ln2_g']), r2(p['ln2_b']),
                      p['Wso'], r2(p['bso']), p['Wa'], r2(p['ba']))
    idx, w = _expand_idx_w(so, aw, reference_points[0])

    sampled = _sc_gather(table4, idx, w).reshape(NQ, C)

    out = _tc_outffn(sampled, p['Wout'], r2(p['bout']), x,
                     r2(p['ln3_g']), r2(p['ln3_b']),
                     p['W1'], r2(p['b1']), p['W2'], r2(p['b2']))
    return out[None]


# speedup vs baseline: 1006.5780x; 1.9056x over previous
import functools

import jax
import jax.numpy as jnp
from jax import lax
from jax.experimental import pallas as pl
from jax.experimental.pallas import tpu as pltpu
from jax.experimental.pallas import tpu_sc as plsc

C = 256
NH = 8
DH = C // NH
NL = 4
NP_ = 4
HGRID = 64
NQ = HGRID * HGRID
NV = NL * NQ
WF = 4

QBLK = 512
NQB = NQ // QBLK

NPAIR = NQ * NH
NCONTRIB = NL * NP_ * 4

NTILE = 32
NPT = NPAIR // NTILE
GB = 16
NBLK = NPT // GB


def _ln(x, g, b):
    m = jnp.mean(x, axis=-1, keepdims=True)
    v = jnp.mean((x - m) ** 2, axis=-1, keepdims=True)
    return (x - m) / jnp.sqrt(v + 1e-5) * g + b


def _qkv_body(x_ref, pos_ref, g_ref, b_ref, w_ref, o_ref):
    xn = _ln(x_ref[...] + pos_ref[...], g_ref[...], b_ref[...])
    res = jnp.dot(xn, w_ref[...], preferred_element_type=jnp.float32)
    for k in range(3 * NH):
        o_ref[k] = res[:, k * DH:(k + 1) * DH]


def _tc_qkv(x, pos, g, b, w):
    return pl.pallas_call(
        _qkv_body,
        grid=(NQB,),
        in_specs=[
            pl.BlockSpec((QBLK, C), lambda i: (i, 0)),
            pl.BlockSpec((QBLK, C), lambda i: (i, 0)),
            pl.BlockSpec((1, C), lambda i: (0, 0)),
            pl.BlockSpec((1, C), lambda i: (0, 0)),
            pl.BlockSpec((C, 3 * C), lambda i: (0, 0)),
        ],
        out_specs=pl.BlockSpec((3 * NH, QBLK, DH), lambda i: (0, i, 0)),
        out_shape=jax.ShapeDtypeStruct((3 * NH, NQ, DH), jnp.float32),
    )(x, pos, g, b, w)


def _attn_body(q_ref, k_ref, v_ref, o_ref):
    q = q_ref[0]
    k = k_ref[0]
    s = lax.dot_general(q, k, (((1,), (1,)), ((), ())),
                        preferred_element_type=jnp.float32) * (DH ** -0.5)
    m = jnp.max(s, axis=-1, keepdims=True)
    e = jnp.exp(s - m)
    z = jnp.sum(e, axis=-1, keepdims=True)
    a = e / z
    o_ref[0] = jnp.dot(a, v_ref[0], preferred_element_type=jnp.float32)


def _tc_attn(qh, kh, vh):
    return pl.pallas_call(
        _attn_body,
        grid=(NH, NQB),
        in_specs=[
            pl.BlockSpec((1, QBLK, DH), lambda h, i: (h, i, 0)),
            pl.BlockSpec((1, NQ, DH), lambda h, i: (h, 0, 0)),
            pl.BlockSpec((1, NQ, DH), lambda h, i: (h, 0, 0)),
        ],
        out_specs=pl.BlockSpec((1, QBLK, DH), lambda h, i: (h, i, 0)),
        out_shape=jax.ShapeDtypeStruct((NH, NQ, DH), jnp.float32),
    )(qh, kh, vh)


def _proj_res_body(a_ref, w_ref, b_ref, r_ref, o_ref):
    a = jnp.concatenate([a_ref[h] for h in range(NH)], axis=-1)
    o_ref[...] = (jnp.dot(a, w_ref[...], preferred_element_type=jnp.float32)
                  + b_ref[...] + r_ref[...])


def _tc_proj_res(attnh, w, b, res):
    return pl.pallas_call(
        _proj_res_body,
        grid=(NQB,),
        in_specs=[
            pl.BlockSpec((NH, QBLK, DH), lambda i: (0, i, 0)),
            pl.BlockSpec((C, C), lambda i: (0, 0)),
            pl.BlockSpec((1, C), lambda i: (0, 0)),
            pl.BlockSpec((QBLK, C), lambda i: (i, 0)),
        ],
        out_specs=pl.BlockSpec((QBLK, C), lambda i: (i, 0)),
        out_shape=jax.ShapeDtypeStruct((NQ, C), jnp.float32),
    )(attnh, w, b, res)


def _value_body(h_ref, g_ref, b_ref, w_ref, bv_ref, o_ref):
    xn = _ln(h_ref[...], g_ref[...], b_ref[...])
    res = jnp.dot(xn, w_ref[...], preferred_element_type=jnp.float32) + bv_ref[...]
    for h in range(NH):
        o_ref[h, 0] = res[:, h * DH:(h + 1) * DH]


def _tc_value(hist, g, b, w, bv):
    blk = 1024
    return pl.pallas_call(
        _value_body,
        grid=(NV // blk,),
        in_specs=[
            pl.BlockSpec((blk, C), lambda i: (i, 0)),
            pl.BlockSpec((1, C), lambda i: (0, 0)),
            pl.BlockSpec((1, C), lambda i: (0, 0)),
            pl.BlockSpec((C, C), lambda i: (0, 0)),
            pl.BlockSpec((1, C), lambda i: (0, 0)),
        ],
        out_specs=pl.BlockSpec((NH, 1, blk, DH), lambda i: (0, i // 4, i % 4, 0)),
        out_shape=jax.ShapeDtypeStruct((NH, NL, NQ, DH), jnp.float32),
    )(hist, g, b, w, bv)


def _corner_body(v_ref, o_ref):
    v = v_ref[0, 0].reshape(HGRID, HGRID, DH)
    sx = jnp.concatenate([v[:, 1:, :], v[:, HGRID - 1:, :]], axis=1)
    sy = jnp.concatenate([v[1:, :, :], v[HGRID - 1:, :, :]], axis=0)
    sxy = jnp.concatenate([sx[1:, :, :], sx[HGRID - 1:, :, :]], axis=0)
    o_ref[0, 0] = jnp.concatenate([v, sx, sy, sxy], axis=-1).reshape(NQ, 4 * DH)


def _tc_corner_pack(vplanes):
    return pl.pallas_call(
        _corner_body,
        grid=(NH, NL),
        in_specs=[pl.BlockSpec((1, 1, NQ, DH), lambda h, l: (h, l, 0, 0))],
        out_specs=pl.BlockSpec((1, 1, NQ, 4 * DH), lambda h, l: (h, l, 0, 0)),
        out_shape=jax.ShapeDtypeStruct((NH, NL, NQ, 4 * DH), jnp.float32),
    )(vplanes)


NLANE = NH * NL * NP_


def _samp_body(x_ref, pos_ref, g_ref, b_ref, wso_ref, bso_ref, wa_ref, ba_ref,
               ref_ref, idx_ref, w_ref):
    xq = _ln(x_ref[...] + pos_ref[...], g_ref[...], b_ref[...])
    so = (jnp.dot(xq, wso_ref[...], preferred_element_type=jnp.float32)
          + bso_ref[...])
    sx = so[:, :NLANE]
    sy = so[:, NLANE:]
    logits = (jnp.dot(xq, wa_ref[...], preferred_element_type=jnp.float32)
              + ba_ref[...])
    parts = []
    for h in range(NH):
        blk = logits[:, h * 16:(h + 1) * 16]
        m = jnp.max(blk, axis=-1, keepdims=True)
        e = jnp.exp(blk - m)
        parts.append(e / jnp.sum(e, axis=-1, keepdims=True))
    aw = jnp.concatenate(parts, axis=-1)

    lane_l = (lax.broadcasted_iota(jnp.int32, (NL, NLANE), 1) // NP_) % NL
    m4 = (lane_l == lax.broadcasted_iota(jnp.int32, (NL, NLANE), 0)
          ).astype(jnp.float32)
    rx = jnp.dot(ref_ref[..., 0], m4, preferred_element_type=jnp.float32)
    ry = jnp.dot(ref_ref[..., 1], m4, preferred_element_type=jnp.float32)

    gx = rx * HGRID + sx - 0.5
    gy = ry * HGRID + sy - 0.5
    x0 = jnp.floor(gx)
    y0 = jnp.floor(gy)
    wx1 = gx - x0
    wx0 = 1.0 - wx1
    wy1 = gy - y0
    wy0 = 1.0 - wy1
    bx = jnp.clip(x0, 0.0, HGRID - 2.0)
    by = jnp.clip(y0, 0.0, HGRID - 2.0)

    lane = lax.broadcasted_iota(jnp.int32, (QBLK, NLANE), 1)
    hl = lane // 16 * NL + (lane // NP_) % NL
    idx_ref[...] = (hl * NQ + by.astype(jnp.int32) * HGRID
                    + bx.astype(jnp.int32))

    rr = lax.broadcasted_iota(jnp.int32, (NLANE, 4 * NLANE), 0)
    cc = lax.broadcasted_iota(jnp.int32, (NLANE, 4 * NLANE), 1)
    acc = jnp.zeros((QBLK, 4 * NLANE), jnp.float32)
    for s, (dy, dx) in enumerate(((0.0, 0.0), (0.0, 1.0), (1.0, 0.0), (1.0, 1.0))):
        sxc = bx + dx
        syc = by + dy
        fx = jnp.where(sxc == x0, wx0, jnp.where(sxc == x0 + 1.0, wx1, 0.0))
        fy = jnp.where(syc == y0, wy0, jnp.where(syc == y0 + 1.0, wy1, 0.0))
        ws = fx * fy * aw
        perm = (cc == (rr // 16) * 64 + s * 16 + rr % 16).astype(jnp.float32)
        acc = acc + jnp.dot(ws, perm, preferred_element_type=jnp.float32)
    w_ref[...] = acc


def _tc_samp(x, pos, g, b, wso, bso, wa, ba, ref):
    return pl.pallas_call(
        _samp_body,
        grid=(NQB,),
        in_specs=[
            pl.BlockSpec((QBLK, C), lambda i: (i, 0)),
            pl.BlockSpec((QBLK, C), lambda i: (i, 0)),
            pl.BlockSpec((1, C), lambda i: (0, 0)),
            pl.BlockSpec((1, C), lambda i: (0, 0)),
            pl.BlockSpec((C, 2 * NLANE), lambda i: (0, 0)),
            pl.BlockSpec((1, 2 * NLANE), lambda i: (0, 0)),
            pl.BlockSpec((C, NLANE), lambda i: (0, 0)),
            pl.BlockSpec((1, NLANE), lambda i: (0, 0)),
            pl.BlockSpec((QBLK, NL, 2), lambda i: (i, 0, 0)),
        ],
        out_specs=[
            pl.BlockSpec((QBLK, NLANE), lambda i: (i, 0)),
            pl.BlockSpec((QBLK, 4 * NLANE), lambda i: (i, 0)),
        ],
        out_shape=[
            jax.ShapeDtypeStruct((NQ, NLANE), jnp.int32),
            jax.ShapeDtypeStruct((NQ, 4 * NLANE), jnp.float32),
        ],
    )(x, pos, g, b, wso, bso, wa, ba, ref)


def _outffn_body(s_ref, wout_ref, bout_ref, x_ref, g_ref, b_ref,
                 w1_ref, b1_ref, w2_ref, b2_ref, o_ref):
    x2 = (jnp.dot(s_ref[...], wout_ref[...], preferred_element_type=jnp.float32)
          + bout_ref[...] + x_ref[...])
    xn = _ln(x2, g_ref[...], b_ref[...])
    h1 = jnp.dot(xn, w1_ref[...], preferred_element_type=jnp.float32) + b1_ref[...]
    ff = jnp.dot(h1, w2_ref[...], preferred_element_type=jnp.float32) + b2_ref[...]
    o_ref[...] = ff + x2


def _tc_outffn(sampled, wout, bout, x, g, b, w1, b1, w2, b2):
    return pl.pallas_call(
        _outffn_body,
        grid=(NQB,),
        in_specs=[
            pl.BlockSpec((QBLK, C), lambda i: (i, 0)),
            pl.BlockSpec((C, C), lambda i: (0, 0)),
            pl.BlockSpec((1, C), lambda i: (0, 0)),
            pl.BlockSpec((QBLK, C), lambda i: (i, 0)),
            pl.BlockSpec((1, C), lambda i: (0, 0)),
            pl.BlockSpec((1, C), lambda i: (0, 0)),
            pl.BlockSpec((C, WF * C), lambda i: (0, 0)),
            pl.BlockSpec((1, WF * C), lambda i: (0, 0)),
            pl.BlockSpec((WF * C, C), lambda i: (0, 0)),
            pl.BlockSpec((1, C), lambda i: (0, 0)),
        ],
        out_specs=pl.BlockSpec((QBLK, C), lambda i: (i, 0)),
        out_shape=jax.ShapeDtypeStruct((NQ, C), jnp.float32),
    )(sampled, wout, bout, x, g, b, w1, b1, w2, b2)


def _sc_body(value_hbm, idx_hbm, w_hbm, out_hbm, idx_v, w_v, rows_v, out_v,
             sem_i, sem_w, sem_r):
    wid = lax.axis_index("s") * 2 + lax.axis_index("c")
    brow = wid * NBLK

    def fire_idx(b, slot):
        pltpu.async_copy(idx_hbm.at[brow + b], idx_v.at[slot], sem_i)
        pltpu.async_copy(w_hbm.at[brow + b], w_v.at[slot], sem_w)

    def wait_idx(slot):
        pltpu.make_async_copy(idx_hbm.at[0], idx_v.at[slot], sem_i).wait()
        pltpu.make_async_copy(w_hbm.at[0], w_v.at[slot], sem_w).wait()

    def fire_gathers(slot):
        def fj(j, c):
            pltpu.async_copy(value_hbm.at[idx_v.at[slot, pl.ds(j * 16, 16)]],
                             rows_v.at[slot, pl.ds(j * 16, 16)], sem_r)
            return c
        lax.fori_loop(0, GB, fj, 0)

    def drain_gathers(slot):
        def dj(j, c):
            pltpu.make_async_copy(
                value_hbm.at[idx_v.at[slot, pl.ds(j * 16, 16)]],
                rows_v.at[slot, pl.ds(j * 16, 16)], sem_r).wait()
            return c
        lax.fori_loop(0, GB, dj, 0)

    def compute_block(b, slot):
        dnums = lax.GatherDimensionNumbers(
            offset_dims=(), collapsed_slice_dims=(0,), start_index_map=(0,))

        def pj(j, c):
            wvecs = [w_v[slot, pl.ds(j * 64 + g * 16, 16)] for g in range(4)]
            acc0 = jnp.zeros((16,), jnp.float32)
            acc1 = jnp.zeros((16,), jnp.float32)
            for i in range(NCONTRIB):
                g, lane = divmod(i, 16)
                ws = lax.gather(
                    wvecs[g], jnp.full((16, 1), lane, jnp.int32), dnums, (1,),
                    mode=lax.GatherScatterMode.PROMISE_IN_BOUNDS)
                r0 = rows_v[slot, j * 16 + i % 16, pl.ds((i // 16) * DH, 16)]
                r1 = rows_v[slot, j * 16 + i % 16, pl.ds((i // 16) * DH + 16, 16)]
                acc0 = acc0 + ws * r0
                acc1 = acc1 + ws * r1
            pit = b * GB + j
            out_v[pit // 4, pl.ds((pit % 4) * DH, 16)] = acc0
            out_v[pit // 4, pl.ds((pit % 4) * DH + 16, 16)] = acc1
            return c
        lax.fori_loop(0, GB, pj, 0)

    def body_seq(b, c):
        fire_idx(b, 0)
        wait_idx(0)
        fire_gathers(0)
        drain_gathers(0)
        compute_block(b, 0)
        return c

    lax.fori_loop(0, NBLK, body_seq, 0)
    pltpu.sync_copy(out_v, out_hbm.at[pl.ds(wid * (NPT // 4), NPT // 4)])


def _sc_gather(valtab, idx, w):
    mesh = plsc.VectorSubcoreMesh(core_axis_name="c", subcore_axis_name="s")
    fn = functools.partial(
        pl.kernel,
        out_type=jax.ShapeDtypeStruct((NPAIR // 4, 4 * DH), jnp.float32),
        mesh=mesh,
        scratch_types=[
            pltpu.VMEM((2, GB * 16), jnp.int32),
            pltpu.VMEM((2, GB * NCONTRIB), jnp.float32),
            pltpu.VMEM((2, GB * 16, 4 * DH), jnp.float32),
            pltpu.VMEM((NPT // 4, 4 * DH), jnp.float32),
            pltpu.SemaphoreType.DMA,
            pltpu.SemaphoreType.DMA,
            pltpu.SemaphoreType.DMA,
        ],
    )(_sc_body)
    return fn(valtab, idx, w)


def kernel(query, history_bevs, reference_points, spatial_shapes,
           level_start_index, pos_embedding, params):
    p = params
    q2 = query[0]
    pos2 = pos_embedding[0]
    hist2 = history_bevs[0]

    def r2(v):
        return v.reshape(1, -1)

    qkvh = _tc_qkv(q2, pos2, r2(p['ln1_g']), r2(p['ln1_b']), p['Wqkv'])
    attnh = _tc_attn(qkvh[:NH], qkvh[NH:2 * NH], qkvh[2 * NH:])
    x = _tc_proj_res(attnh, p['Wo'], r2(p['bo']), q2)

    vplanes = _tc_value(hist2, r2(p['ln2_g']), r2(p['ln2_b']), p['Wv'],
                        r2(p['bv']))
    table4 = _tc_corner_pack(vplanes).reshape(NH * NL * NQ, 4 * DH)

    wso_p = jnp.concatenate([p['Wso'][:, 0::2], p['Wso'][:, 1::2]], axis=1)
    bso_p = jnp.concatenate([p['bso'][0::2], p['bso'][1::2]])
    idxq, wq = _tc_samp(x, pos2, r2(p['ln2_g']), r2(p['ln2_b']),
                        wso_p, r2(bso_p), p['Wa'], r2(p['ba']),
                        reference_points[0])

    sampled = _sc_gather(table4, idxq.reshape(NPAIR // GB, GB * 16),
                         wq.reshape(NPAIR // GB, GB * NCONTRIB)).reshape(NQ, C)

    out = _tc_outffn(sampled, p['Wout'], r2(p['bout']), x,
                     r2(p['ln3_g']), r2(p['ln3_b']),
                     p['W1'], r2(p['b1']), p['W2'], r2(p['b2']))
    return out[None]
